# Initial kernel scaffold; baseline (speedup 1.0000x reference)
import functools

import jax
import jax.numpy as jnp
from jax import lax
from jax.experimental import pallas as pl
from jax.experimental.pallas import tpu as pltpu
from jax.experimental.pallas import tpu_sc as plsc

_N = 10000
_E = 320000
_C = 128
_G = 64
_K = 128
_NCHUNK = _E // _K
_NTILE = 16
_NW = 2 * _NTILE
_RPT = _N // _NTILE
_ZR = 125

_mesh = plsc.VectorSubcoreMesh(core_axis_name="c", subcore_axis_name="s")


@functools.partial(
    pl.kernel,
    out_type=jax.ShapeDtypeStruct((2, _N, 16), jnp.float32),
    mesh=_mesh,
    scratch_types=[
        pltpu.VMEM((_K,), jnp.int32),
        pltpu.VMEM((_K, 16), jnp.float32),
        pltpu.VMEM((_RPT, 16), jnp.float32),
        pltpu.VMEM_SHARED((_N, 16), jnp.float32),
        pltpu.SemaphoreType.DMA,
    ],
)
def _deg_kernel(dst_hbm, out_hbm, dst_v, ones_v, zbuf, acc, sem):
    cid = lax.axis_index("c")
    sid = lax.axis_index("s")
    wid = sid * 2 + cid

    one = jnp.ones((16,), jnp.float32)
    zero = jnp.zeros((16,), jnp.float32)

    def fill_ones(i, _):
        ones_v[i, pl.ds(0, 16)] = one
        return 0
    lax.fori_loop(0, _K, fill_ones, 0)

    def fill_zeros(i, _):
        zbuf[i, pl.ds(0, 16)] = zero
        return 0
    lax.fori_loop(0, _RPT, fill_zeros, 0)

    row0 = sid * _RPT
    pltpu.sync_copy(zbuf, acc.at[pl.ds(row0, _RPT)])
    plsc.subcore_barrier()

    niter = (_NCHUNK + _NW - 1) // _NW

    def body(i, _):
        c = i * _NW + wid

        @pl.when(c < _NCHUNK)
        def _():
            pltpu.sync_copy(dst_hbm.at[pl.ds(c * _K, _K)], dst_v)
            pltpu.sync_copy(ones_v, acc.at[dst_v], add=True)
        return 0
    lax.fori_loop(0, niter, body, 0)
    plsc.subcore_barrier()

    pltpu.sync_copy(acc.at[pl.ds(row0, _RPT)],
                    out_hbm.at[cid, pl.ds(row0, _RPT)])


@functools.partial(
    pl.kernel,
    out_type=jax.ShapeDtypeStruct((2, _N, _C), jnp.float32),
    mesh=_mesh,
    scratch_types=[
        pltpu.VMEM((_K,), jnp.int32),
        pltpu.VMEM((_K,), jnp.int32),
        pltpu.VMEM((_K, _C), jnp.float32),
        pltpu.VMEM((_ZR, _C), jnp.float32),
        pltpu.VMEM_SHARED((_N, _C), jnp.float32),
        pltpu.SemaphoreType.DMA,
    ],
)
def _agg_kernel(h_hbm, src_hbm, dst_hbm, out_hbm,
                src_v, dst_v, rows_v, zbuf, acc, sem):
    cid = lax.axis_index("c")
    sid = lax.axis_index("s")
    wid = sid * 2 + cid

    zero = jnp.zeros((16,), jnp.float32)

    def zrow(i, _):
        def zcol(j, _):
            zbuf[i, pl.ds(j * 16, 16)] = zero
            return 0
        lax.fori_loop(0, _C // 16, zcol, 0)
        return 0
    lax.fori_loop(0, _ZR, zrow, 0)

    row0 = sid * _RPT
    for k in range(_RPT // _ZR):
        pltpu.sync_copy(zbuf, acc.at[pl.ds(row0 + k * _ZR, _ZR)])
    plsc.subcore_barrier()

    niter = (_NCHUNK + _NW - 1) // _NW

    def body(i, _):
        c = i * _NW + wid

        @pl.when(c < _NCHUNK)
        def _():
            base = c * _K
            pltpu.sync_copy(src_hbm.at[pl.ds(base, _K)], src_v)
            pltpu.sync_copy(dst_hbm.at[pl.ds(base, _K)], dst_v)
            pltpu.async_copy(h_hbm.at[src_v], rows_v, sem).wait()
            pltpu.sync_copy(rows_v, acc.at[dst_v], add=True)
        return 0
    lax.fori_loop(0, niter, body, 0)
    plsc.subcore_barrier()

    for k in range(_RPT // _ZR):
        r = row0 + k * _ZR
        pltpu.sync_copy(acc.at[pl.ds(r, _ZR)], out_hbm.at[cid, pl.ds(r, _ZR)])


_NB = 1000
_NBLK = _N // _NB


def _t1_body(deg_ref, x_ref, w_ref, dinv_ref, h_ref):
    deg = deg_ref[0, :, 0:1] + deg_ref[1, :, 0:1] + 1.0
    dinv = lax.rsqrt(deg)
    dinv_ref[...] = dinv
    h_ref[...] = jnp.dot(x_ref[...], w_ref[...],
                         preferred_element_type=jnp.float32) * dinv


_t1 = pl.pallas_call(
    _t1_body,
    grid=(_NBLK,),
    in_specs=[
        pl.BlockSpec((2, _NB, 16), lambda i: (0, i, 0)),
        pl.BlockSpec((_NB, _C), lambda i: (i, 0)),
        pl.BlockSpec((_C, _C), lambda i: (0, 0)),
    ],
    out_specs=[
        pl.BlockSpec((_NB, 1), lambda i: (i, 0)),
        pl.BlockSpec((_NB, _C), lambda i: (i, 0)),
    ],
    out_shape=[
        jax.ShapeDtypeStruct((_N, 1), jnp.float32),
        jax.ShapeDtypeStruct((_N, _C), jnp.float32),
    ],
)


def _tmid_body(p_ref, hprev_ref, dinv_ref, b_ref, w_ref, h_ref):
    dinv = dinv_ref[...]
    agg = p_ref[0] + p_ref[1] + hprev_ref[...]
    x = jnp.maximum(dinv * agg + b_ref[...], 0.0)
    h_ref[...] = jnp.dot(x, w_ref[...],
                         preferred_element_type=jnp.float32) * dinv


_tmid = pl.pallas_call(
    _tmid_body,
    grid=(_NBLK,),
    in_specs=[
        pl.BlockSpec((2, _NB, _C), lambda i: (0, i, 0)),
        pl.BlockSpec((_NB, _C), lambda i: (i, 0)),
        pl.BlockSpec((_NB, 1), lambda i: (i, 0)),
        pl.BlockSpec((1, _C), lambda i: (0, 0)),
        pl.BlockSpec((_C, _C), lambda i: (0, 0)),
    ],
    out_specs=pl.BlockSpec((_NB, _C), lambda i: (i, 0)),
    out_shape=jax.ShapeDtypeStruct((_N, _C), jnp.float32),
)


def _t6_body(p_ref, hprev_ref, dinv_ref, b_ref, batch_ref,
             fc1w_ref, fc1b_ref, fc2w_ref, fc2b_ref,
             out_ref, pool_acc, cnt_acc):
    i = pl.program_id(0)

    @pl.when(i == 0)
    def _():
        pool_acc[...] = jnp.zeros_like(pool_acc)
        cnt_acc[...] = jnp.zeros_like(cnt_acc)

    dinv = dinv_ref[...]
    agg = p_ref[0] + p_ref[1] + hprev_ref[...]
    x = jnp.maximum(dinv * agg + b_ref[...], 0.0)

    bt = batch_ref[0, :]
    onehot = (bt[None, :] ==
              lax.broadcasted_iota(jnp.int32, (_G, _NB), 0)
              ).astype(jnp.float32)
    pool_acc[...] += jnp.dot(onehot, x, preferred_element_type=jnp.float32)
    cnt_acc[...] += jnp.sum(onehot, axis=1, keepdims=True)

    @pl.when(i == _NBLK - 1)
    def _():
        pooled = pool_acc[...] / jnp.maximum(cnt_acc[...], 1.0)
        g = jnp.maximum(
            jnp.dot(pooled, fc1w_ref[...],
                    preferred_element_type=jnp.float32) + fc1b_ref[...], 0.0)
        out_ref[...] = jnp.dot(
            g, fc2w_ref[...], preferred_element_type=jnp.float32) + fc2b_ref[...]


_t6 = pl.pallas_call(
    _t6_body,
    grid=(_NBLK,),
    in_specs=[
        pl.BlockSpec((2, _NB, _C), lambda i: (0, i, 0)),
        pl.BlockSpec((_NB, _C), lambda i: (i, 0)),
        pl.BlockSpec((_NB, 1), lambda i: (i, 0)),
        pl.BlockSpec((1, _C), lambda i: (0, 0)),
        pl.BlockSpec((1, _NB), lambda i: (0, i)),
        pl.BlockSpec((_C, _C), lambda i: (0, 0)),
        pl.BlockSpec((1, _C), lambda i: (0, 0)),
        pl.BlockSpec((_C, _C), lambda i: (0, 0)),
        pl.BlockSpec((1, _C), lambda i: (0, 0)),
    ],
    out_specs=pl.BlockSpec((_G, _C), lambda i: (0, 0)),
    out_shape=jax.ShapeDtypeStruct((_G, _C), jnp.float32),
    scratch_shapes=[
        pltpu.VMEM((_G, _C), jnp.float32),
        pltpu.VMEM((_G, 1), jnp.float32),
    ],
)


@jax.jit
def kernel(x, edge_index, batch,
           W1, b1, W2, b2, W3, b3, W4, b4, W5, b5,
           fc1_W, fc1_b, fc2_W, fc2_b):
    src = edge_index[0]
    dst = edge_index[1]

    degp = _deg_kernel(dst)
    dinv, h = _t1(degp, x, W1)

    for (b_prev, w_next) in ((b1, W2), (b2, W3), (b3, W4), (b4, W5)):
        p = _agg_kernel(h, src, dst)
        h = _tmid(p, h, dinv, b_prev[None, :], w_next)

    p = _agg_kernel(h, src, dst)
    out = _t6(p, h, dinv, b5[None, :], batch[None, :],
              fc1_W, fc1_b[None, :], fc2_W, fc2_b[None, :])
    return out


# speedup vs baseline: 11.7525x; 11.7525x over previous
import functools

import jax
import jax.numpy as jnp
from jax import lax
from jax.experimental import pallas as pl
from jax.experimental.pallas import tpu as pltpu
from jax.experimental.pallas import tpu_sc as plsc

_N = 10000
_E = 320000
_C = 128
_G = 64
_K = 128
_NCHUNK = _E // _K
_NTILE = 16
_NW = 2 * _NTILE
_NP = 10240
_RPT = _NP // _NTILE
_ZR = 128

_mesh = plsc.VectorSubcoreMesh(core_axis_name="c", subcore_axis_name="s")


@functools.partial(
    pl.kernel,
    out_type=jax.ShapeDtypeStruct((2, _NP, _C), jnp.float32),
    mesh=_mesh,
    scratch_types=[
        pltpu.VMEM((_K,), jnp.int32),
        pltpu.VMEM((_K, _C), jnp.float32),
        pltpu.VMEM((_ZR, _C), jnp.float32),
        pltpu.VMEM_SHARED((_NP, _C), jnp.float32),
        pltpu.SemaphoreType.DMA,
    ],
)
def _deg_kernel(dst_hbm, out_hbm, dst_v, ones_v, zbuf, acc, sem):
    cid = lax.axis_index("c")
    sid = lax.axis_index("s")
    wid = sid * 2 + cid

    zero = jnp.zeros((16,), jnp.float32)
    one = jnp.ones((16,), jnp.float32)

    def orow(i, _):
        def ocol(j, _):
            ones_v[i, pl.ds(j * 16, 16)] = one
            return 0
        lax.fori_loop(0, _C // 16, ocol, 0)
        return 0
    lax.fori_loop(0, _K, orow, 0)

    def zrow(i, _):
        def zcol(j, _):
            zbuf[i, pl.ds(j * 16, 16)] = zero
            return 0
        lax.fori_loop(0, _C // 16, zcol, 0)
        return 0
    lax.fori_loop(0, _ZR, zrow, 0)

    row0 = sid * _RPT
    for k in range(_RPT // _ZR):
        pltpu.sync_copy(zbuf, acc.at[pl.ds(row0 + k * _ZR, _ZR)])
    plsc.subcore_barrier()

    niter = (_NCHUNK + _NW - 1) // _NW

    def body(i, _):
        c = i * _NW + wid

        @pl.when(c < _NCHUNK)
        def _():
            pltpu.sync_copy(dst_hbm.at[pl.ds(c * _K, _K)], dst_v)
            pltpu.sync_copy(ones_v, acc.at[dst_v], add=True)
        return 0
    lax.fori_loop(0, niter, body, 0)
    plsc.subcore_barrier()

    for k in range(_RPT // _ZR):
        r = row0 + k * _ZR
        pltpu.sync_copy(acc.at[pl.ds(r, _ZR)], out_hbm.at[cid, pl.ds(r, _ZR)])


@functools.partial(
    pl.kernel,
    out_type=jax.ShapeDtypeStruct((2, _NP, _C), jnp.float32),
    mesh=_mesh,
    scratch_types=[
        pltpu.VMEM((_K,), jnp.int32),
        pltpu.VMEM((_K,), jnp.int32),
        pltpu.VMEM((_K, _C), jnp.float32),
        pltpu.VMEM((_ZR, _C), jnp.float32),
        pltpu.VMEM_SHARED((_NP, _C), jnp.float32),
        pltpu.SemaphoreType.DMA,
    ],
)
def _agg_kernel(h_hbm, src_hbm, dst_hbm, out_hbm,
                src_v, dst_v, rows_v, zbuf, acc, sem):
    cid = lax.axis_index("c")
    sid = lax.axis_index("s")
    wid = sid * 2 + cid

    zero = jnp.zeros((16,), jnp.float32)

    def zrow(i, _):
        def zcol(j, _):
            zbuf[i, pl.ds(j * 16, 16)] = zero
            return 0
        lax.fori_loop(0, _C // 16, zcol, 0)
        return 0
    lax.fori_loop(0, _ZR, zrow, 0)

    row0 = sid * _RPT
    for k in range(_RPT // _ZR):
        pltpu.sync_copy(zbuf, acc.at[pl.ds(row0 + k * _ZR, _ZR)])
    plsc.subcore_barrier()

    niter = (_NCHUNK + _NW - 1) // _NW

    def body(i, _):
        c = i * _NW + wid

        @pl.when(c < _NCHUNK)
        def _():
            base = c * _K
            pltpu.sync_copy(src_hbm.at[pl.ds(base, _K)], src_v)
            pltpu.sync_copy(dst_hbm.at[pl.ds(base, _K)], dst_v)
            pltpu.async_copy(h_hbm.at[src_v], rows_v, sem).wait()
            pltpu.sync_copy(rows_v, acc.at[dst_v], add=True)
        return 0
    lax.fori_loop(0, niter, body, 0)
    plsc.subcore_barrier()

    for k in range(_RPT // _ZR):
        r = row0 + k * _ZR
        pltpu.sync_copy(acc.at[pl.ds(r, _ZR)], out_hbm.at[cid, pl.ds(r, _ZR)])


_NB = 1024
_NBLK = _NP // _NB


def _t1_body(deg_ref, x_ref, w_ref, dinv_ref, h_ref):
    deg = deg_ref[0, :, 0:1] + deg_ref[1, :, 0:1] + 1.0
    dinv = lax.rsqrt(deg)
    dinv_ref[...] = dinv
    h_ref[...] = jnp.dot(x_ref[...], w_ref[...],
                         preferred_element_type=jnp.float32) * dinv


_t1 = pl.pallas_call(
    _t1_body,
    grid=(_NBLK,),
    in_specs=[
        pl.BlockSpec((2, _NB, _C), lambda i: (0, i, 0)),
        pl.BlockSpec((_NB, _C), lambda i: (i, 0)),
        pl.BlockSpec((_C, _C), lambda i: (0, 0)),
    ],
    out_specs=[
        pl.BlockSpec((_NB, 1), lambda i: (i, 0)),
        pl.BlockSpec((_NB, _C), lambda i: (i, 0)),
    ],
    out_shape=[
        jax.ShapeDtypeStruct((_NP, 1), jnp.float32),
        jax.ShapeDtypeStruct((_NP, _C), jnp.float32),
    ],
)


def _tmid_body(p_ref, hprev_ref, dinv_ref, b_ref, w_ref, h_ref):
    dinv = dinv_ref[...]
    agg = p_ref[0] + p_ref[1] + hprev_ref[...]
    x = jnp.maximum(dinv * agg + b_ref[...], 0.0)
    h_ref[...] = jnp.dot(x, w_ref[...],
                         preferred_element_type=jnp.float32) * dinv


_tmid = pl.pallas_call(
    _tmid_body,
    grid=(_NBLK,),
    in_specs=[
        pl.BlockSpec((2, _NB, _C), lambda i: (0, i, 0)),
        pl.BlockSpec((_NB, _C), lambda i: (i, 0)),
        pl.BlockSpec((_NB, 1), lambda i: (i, 0)),
        pl.BlockSpec((1, _C), lambda i: (0, 0)),
        pl.BlockSpec((_C, _C), lambda i: (0, 0)),
    ],
    out_specs=pl.BlockSpec((_NB, _C), lambda i: (i, 0)),
    out_shape=jax.ShapeDtypeStruct((_NP, _C), jnp.float32),
)


def _t6_body(p_ref, hprev_ref, dinv_ref, b_ref, batch_ref,
             fc1w_ref, fc1b_ref, fc2w_ref, fc2b_ref,
             out_ref, pool_acc, cnt_acc):
    i = pl.program_id(0)

    @pl.when(i == 0)
    def _():
        pool_acc[...] = jnp.zeros_like(pool_acc)
        cnt_acc[...] = jnp.zeros_like(cnt_acc)

    dinv = dinv_ref[...]
    agg = p_ref[0] + p_ref[1] + hprev_ref[...]
    x = jnp.maximum(dinv * agg + b_ref[...], 0.0)

    bt = batch_ref[0, 0, :]
    onehot = (bt[None, :] ==
              lax.broadcasted_iota(jnp.int32, (_G, _NB), 0)
              ).astype(jnp.float32)
    pool_acc[...] += jnp.dot(onehot, x, preferred_element_type=jnp.float32)
    cnt_acc[...] += jnp.sum(onehot, axis=1, keepdims=True)

    @pl.when(i == _NBLK - 1)
    def _():
        pooled = pool_acc[...] / jnp.maximum(cnt_acc[...], 1.0)
        g = jnp.maximum(
            jnp.dot(pooled, fc1w_ref[...],
                    preferred_element_type=jnp.float32) + fc1b_ref[...], 0.0)
        out_ref[...] = jnp.dot(
            g, fc2w_ref[...], preferred_element_type=jnp.float32) + fc2b_ref[...]


_t6 = pl.pallas_call(
    _t6_body,
    grid=(_NBLK,),
    in_specs=[
        pl.BlockSpec((2, _NB, _C), lambda i: (0, i, 0)),
        pl.BlockSpec((_NB, _C), lambda i: (i, 0)),
        pl.BlockSpec((_NB, 1), lambda i: (i, 0)),
        pl.BlockSpec((1, _C), lambda i: (0, 0)),
        pl.BlockSpec((1, 1, _NB), lambda i: (i, 0, 0)),
        pl.BlockSpec((_C, _C), lambda i: (0, 0)),
        pl.BlockSpec((1, _C), lambda i: (0, 0)),
        pl.BlockSpec((_C, _C), lambda i: (0, 0)),
        pl.BlockSpec((1, _C), lambda i: (0, 0)),
    ],
    out_specs=pl.BlockSpec((_G, _C), lambda i: (0, 0)),
    out_shape=jax.ShapeDtypeStruct((_G, _C), jnp.float32),
    scratch_shapes=[
        pltpu.VMEM((_G, _C), jnp.float32),
        pltpu.VMEM((_G, 1), jnp.float32),
    ],
)


@jax.jit
def kernel(x, edge_index, batch,
           W1, b1, W2, b2, W3, b3, W4, b4, W5, b5,
           fc1_W, fc1_b, fc2_W, fc2_b):
    src = edge_index[0]
    dst = edge_index[1]
    xp = jnp.pad(x, ((0, _NP - _N), (0, 0)))
    bp = jnp.pad(batch, (0, _NP - _N), constant_values=_G)

    degp = _deg_kernel(dst)
    dinv, h = _t1(degp, xp, W1)

    for (b_prev, w_next) in ((b1, W2), (b2, W3), (b3, W4), (b4, W5)):
        p = _agg_kernel(h, src, dst)
        h = _tmid(p, h, dinv, b_prev[None, :], w_next)

    p = _agg_kernel(h, src, dst)
    out = _t6(p, h, dinv, b5[None, :], bp.reshape(_NBLK, 1, _NB),
              fc1_W, fc1_b[None, :], fc2_W, fc2_b[None, :])
    return out
